# baseline (speedup 1.0000x reference)
import functools

import jax
import jax.numpy as jnp
from jax import lax
from jax.experimental import pallas as pl
from jax.experimental.pallas import tpu as pltpu
from jax.experimental.pallas import tpu_sc as plsc

NC, NS, LANES = 2, 16, 16
NW = NC * NS
NB = 4
C0 = 128


@jax.jit
def _lookup(ids_flat, seg_flat, wtab, ptab, stab):
    N, = ids_flat.shape
    V, E = wtab.shape
    P = ptab.shape[0]
    S = stab.shape[0]
    FB = E // LANES
    PE = P * E
    C = P
    C1 = C - C0
    n_per_w = N // NW
    n_chunks = n_per_w // C
    assert n_chunks >= NB and C1 <= 128 and C0 % 8 == 0
    mesh = plsc.VectorSubcoreMesh(core_axis_name="c", subcore_axis_name="s")

    @functools.partial(
        pl.kernel,
        mesh=mesh,
        out_type=jax.ShapeDtypeStruct((N, E), jnp.float32),
        compiler_params=pltpu.CompilerParams(
            needs_layout_passes=False, use_tc_tiling_on_sc=False),
        scratch_types=[
            pltpu.VMEM((NB, 2, C0), jnp.int32),
            pltpu.VMEM((NB, C), jnp.int32),
            pltpu.VMEM((NB, C, E), jnp.float32),
            pltpu.VMEM((P, E), jnp.float32),
            pltpu.VMEM((S, E), jnp.float32),
            pltpu.VMEM((S * P * E,), jnp.float32),
            [pltpu.SemaphoreType.DMA] * NB,
            [pltpu.SemaphoreType.DMA] * NB,
            [pltpu.SemaphoreType.DMA] * NB,
        ],
    )
    def lookup(ids_hbm, seg_hbm, wtab_hbm, ptab_hbm, stab_hbm, out_hbm,
               idx_v, segc_v, rows_v, ptab_v, stab_v, comb_v,
               isems, gsems, osems):
        wid = lax.axis_index("s") * NC + lax.axis_index("c")
        base = wid * n_per_w
        iota = lax.iota(jnp.int32, LANES)

        pltpu.sync_copy(ptab_hbm, ptab_v)
        pltpu.sync_copy(stab_hbm, stab_v)
        for s in range(S):
            for fb in range(FB):
                fsl = pl.ds(fb * LANES, LANES)

                def build_body(t, carry, s=s, fb=fb, fsl=fsl):
                    comb_v[pl.ds(s * PE + t * E + fb * LANES, LANES)] = (
                        ptab_v[t, fsl] + stab_v[s, fsl])
                    return carry

                lax.fori_loop(0, P, build_body, 0)

        def fire_idx(j, b):
            start = base + j * C
            pltpu.async_copy(ids_hbm.at[pl.ds(start, C0)],
                             idx_v.at[b, 0], isems[b])
            pltpu.async_copy(ids_hbm.at[pl.ds(start + C0, C1)],
                             idx_v.at[b, 1, pl.ds(0, C1)], isems[b])
            pltpu.async_copy(seg_hbm.at[pl.ds(start, C)],
                             segc_v.at[b], isems[b])

        def wait_idx(j, b):
            start = base + j * C
            pltpu.make_async_copy(ids_hbm.at[pl.ds(start, C0)],
                                  idx_v.at[b, 0], isems[b]).wait()
            pltpu.make_async_copy(ids_hbm.at[pl.ds(start + C0, C1)],
                                  idx_v.at[b, 1, pl.ds(0, C1)],
                                  isems[b]).wait()
            pltpu.make_async_copy(seg_hbm.at[pl.ds(start, C)],
                                  segc_v.at[b], isems[b]).wait()

        def fire_gather(b):
            pltpu.async_copy(wtab_hbm.at[idx_v.at[b, 0]],
                             rows_v.at[b, pl.ds(0, C0)], gsems[b])
            pltpu.async_copy(wtab_hbm.at[idx_v.at[b, 1, pl.ds(0, C1)]],
                             rows_v.at[b, pl.ds(C0, C1)], gsems[b])

        def wait_gather(b):
            pltpu.make_async_copy(wtab_hbm.at[idx_v.at[b, 0]],
                                  rows_v.at[b, pl.ds(0, C0)],
                                  gsems[b]).wait()
            pltpu.make_async_copy(wtab_hbm.at[idx_v.at[b, 1, pl.ds(0, C1)]],
                                  rows_v.at[b, pl.ds(C0, C1)],
                                  gsems[b]).wait()

        def fire_scatter(j, b):
            start = base + j * C
            pltpu.async_copy(rows_v.at[b], out_hbm.at[pl.ds(start, C)],
                             osems[b])

        def wait_scatter(j, b):
            start = base + j * C
            pltpu.make_async_copy(
                rows_v.at[b], out_hbm.at[pl.ds(start, C)], osems[b]).wait()

        for b in range(NB):
            fire_idx(b, b)
        for b in range(NB - 1):
            wait_idx(b, b)
            fire_gather(b)

        def chunk_body(i, carry):
            for bb in range(NB):

                @pl.when(lax.rem(i, NB) == bb)
                def _process(bb=bb):
                    wait_gather(bb)
                    b2 = (bb + NB - 1) % NB

                    @pl.when(i + NB - 1 < n_chunks)
                    def _refill():
                        @pl.when(i >= 1)
                        def _():
                            wait_scatter(i - 1, b2)

                        wait_idx_dyn(i + NB - 1, b2)
                        fire_gather(b2)

                    def tok_body(t, tvec):
                        svec = plsc.load_gather(segc_v.at[bb], [tvec])
                        bidx = svec * PE + tvec * E
                        for fb in range(FB):
                            cidx = bidx + (fb * LANES + iota)
                            add = plsc.load_gather(comb_v, [cidx])
                            fsl = pl.ds(fb * LANES, LANES)
                            rows_v[bb, t, fsl] = rows_v[bb, t, fsl] + add
                        return tvec + 1

                    plsc.parallel_loop(
                        0, C, unroll=4, carry=iota * 0)(tok_body)

                    fire_scatter(i, bb)
                    @pl.when(i + NB < n_chunks)
                    def _():
                        fire_idx_dyn(i + NB, bb)

            return carry

        fire_idx_dyn = fire_idx
        wait_idx_dyn = wait_idx
        lax.fori_loop(0, n_chunks, chunk_body, 0)

        for j in range(n_chunks - NB, n_chunks):
            wait_scatter(j, j % NB)

    return lookup(ids_flat, seg_flat, wtab, ptab, stab)


def kernel(input_ids, segment_ids, word_embeddings, position_embeddings,
           segment_embeddings):
    B, L = input_ids.shape
    E = word_embeddings.shape[1]
    N = B * L
    ids_flat = input_ids.reshape(N).astype(jnp.int32)
    seg_flat = segment_ids.reshape(N).astype(jnp.int32)
    out = _lookup(ids_flat, seg_flat, word_embeddings,
                  position_embeddings, segment_embeddings)
    return out.reshape(B, L, E)


# speedup vs baseline: 1.0596x; 1.0596x over previous
import functools

import jax
import jax.numpy as jnp
from jax import lax
from jax.experimental import pallas as pl
from jax.experimental.pallas import tpu as pltpu
from jax.experimental.pallas import tpu_sc as plsc

NC, NS, LANES = 2, 16, 16
NW = NC * NS
NB = 4
C0 = 128


@jax.jit
def _lookup(ids_flat, seg_flat, wtab, ptab, stab):
    N, = ids_flat.shape
    V, E = wtab.shape
    P = ptab.shape[0]
    S = stab.shape[0]
    FB = E // LANES
    PE = P * E
    C = P
    C1 = C - C0
    n_per_w = N // NW
    n_chunks = n_per_w // C
    assert n_chunks >= NB and C1 <= 128 and C0 % 8 == 0
    mesh = plsc.VectorSubcoreMesh(core_axis_name="c", subcore_axis_name="s")

    @functools.partial(
        pl.kernel,
        mesh=mesh,
        out_type=jax.ShapeDtypeStruct((N, E), jnp.float32),
        compiler_params=pltpu.CompilerParams(
            needs_layout_passes=False, use_tc_tiling_on_sc=False),
        scratch_types=[
            pltpu.VMEM((NB, 2, C0), jnp.int32),
            pltpu.VMEM((NB, C), jnp.int32),
            pltpu.VMEM((NB, C, E), jnp.float32),
            pltpu.VMEM((P, E), jnp.float32),
            pltpu.VMEM((S, E), jnp.float32),
            pltpu.VMEM((S * P * E,), jnp.float32),
            [pltpu.SemaphoreType.DMA] * NB,
            [pltpu.SemaphoreType.DMA] * NB,
            [pltpu.SemaphoreType.DMA] * NB,
        ],
    )
    def lookup(ids_hbm, seg_hbm, wtab_hbm, ptab_hbm, stab_hbm, out_hbm,
               idx_v, segc_v, rows_v, ptab_v, stab_v, comb_v,
               isems, gsems, osems):
        wid = lax.axis_index("s") * NC + lax.axis_index("c")
        base = wid * n_per_w
        iota = lax.iota(jnp.int32, LANES)

        pltpu.sync_copy(ptab_hbm, ptab_v)
        pltpu.sync_copy(stab_hbm, stab_v)
        for s in range(S):
            for fb in range(FB):
                fsl = pl.ds(fb * LANES, LANES)

                def build_body(t, carry, s=s, fb=fb, fsl=fsl):
                    comb_v[pl.ds(s * PE + t * E + fb * LANES, LANES)] = (
                        ptab_v[t, fsl] + stab_v[s, fsl])
                    return carry

                lax.fori_loop(0, P, build_body, 0)

        def fire_idx(j, b):
            start = base + j * C
            pltpu.async_copy(ids_hbm.at[pl.ds(start, C0)],
                             idx_v.at[b, 0], isems[b])
            pltpu.async_copy(ids_hbm.at[pl.ds(start + C0, C1)],
                             idx_v.at[b, 1, pl.ds(0, C1)], isems[b])
            pltpu.async_copy(seg_hbm.at[pl.ds(start, C)],
                             segc_v.at[b], isems[b])

        def wait_idx(j, b):
            start = base + j * C
            pltpu.make_async_copy(ids_hbm.at[pl.ds(start, C0)],
                                  idx_v.at[b, 0], isems[b]).wait()
            pltpu.make_async_copy(ids_hbm.at[pl.ds(start + C0, C1)],
                                  idx_v.at[b, 1, pl.ds(0, C1)],
                                  isems[b]).wait()
            pltpu.make_async_copy(seg_hbm.at[pl.ds(start, C)],
                                  segc_v.at[b], isems[b]).wait()

        def fire_gather(b):
            pltpu.async_copy(wtab_hbm.at[idx_v.at[b, 0]],
                             rows_v.at[b, pl.ds(0, C0)], gsems[b])
            pltpu.async_copy(wtab_hbm.at[idx_v.at[b, 1, pl.ds(0, C1)]],
                             rows_v.at[b, pl.ds(C0, C1)], gsems[b])

        def wait_gather(b):
            pltpu.make_async_copy(wtab_hbm.at[idx_v.at[b, 0]],
                                  rows_v.at[b, pl.ds(0, C0)],
                                  gsems[b]).wait()
            pltpu.make_async_copy(wtab_hbm.at[idx_v.at[b, 1, pl.ds(0, C1)]],
                                  rows_v.at[b, pl.ds(C0, C1)],
                                  gsems[b]).wait()

        def fire_scatter(j, b):
            start = base + j * C
            pltpu.async_copy(rows_v.at[b], out_hbm.at[pl.ds(start, C)],
                             osems[b])

        def wait_scatter(j, b):
            start = base + j * C
            pltpu.make_async_copy(
                rows_v.at[b], out_hbm.at[pl.ds(start, C)], osems[b]).wait()

        for b in range(NB):
            fire_idx(b, b)
        for b in range(NB - 1):
            wait_idx(b, b)
            fire_gather(b)

        def chunk_body(i, carry):
            for bb in range(NB):

                @pl.when(lax.rem(i, NB) == bb)
                def _process(bb=bb):
                    wait_gather(bb)

                    def tok_body(t, tvec):
                        svec = plsc.load_gather(segc_v.at[bb], [tvec])
                        bidx = svec * PE + tvec * E
                        for fb in range(FB):
                            cidx = bidx + (fb * LANES + iota)
                            add = plsc.load_gather(comb_v, [cidx])
                            fsl = pl.ds(fb * LANES, LANES)
                            rows_v[bb, t, fsl] = rows_v[bb, t, fsl] + add
                        return tvec + 1

                    plsc.parallel_loop(
                        0, C, unroll=4, carry=iota * 0)(tok_body)

                    fire_scatter(i, bb)
                    @pl.when(i + NB < n_chunks)
                    def _():
                        fire_idx_dyn(i + NB, bb)

                    b2 = (bb + NB - 1) % NB

                    @pl.when(i + NB - 1 < n_chunks)
                    def _refill():
                        wait_idx_dyn(i + NB - 1, b2)

                        @pl.when(i >= 1)
                        def _():
                            wait_scatter(i - 1, b2)

                        fire_gather(b2)

            return carry

        fire_idx_dyn = fire_idx
        wait_idx_dyn = wait_idx
        lax.fori_loop(0, n_chunks, chunk_body, 0)

        for j in range(n_chunks - NB, n_chunks):
            wait_scatter(j, j % NB)

    return lookup(ids_flat, seg_flat, wtab, ptab, stab)


def kernel(input_ids, segment_ids, word_embeddings, position_embeddings,
           segment_embeddings):
    B, L = input_ids.shape
    E = word_embeddings.shape[1]
    N = B * L
    ids_flat = input_ids.reshape(N).astype(jnp.int32)
    seg_flat = segment_ids.reshape(N).astype(jnp.int32)
    out = _lookup(ids_flat, seg_flat, word_embeddings,
                  position_embeddings, segment_embeddings)
    return out.reshape(B, L, E)
